# baseline (speedup 1.0000x reference)
import jax
import jax.numpy as jnp
import numpy as np
from jax import lax
from jax.experimental import pallas as pl
from jax.experimental.pallas import tpu as pltpu
from jax.experimental.pallas import tpu_sc as plsc

_VOCAB = 1000000
_EMB = 64
_BATCH = 16
_SEQ = 2048

_NC, _NS, _L = 2, 16, 16
_NW = _NC * _NS
_PER_W = _BATCH * _SEQ // _NW
_C = 128
_NCHUNK = _PER_W // _C


def _pos_table(emb, seq):
    enc = np.zeros((seq, emb), dtype=np.float32)
    pos = np.arange(0.0, seq, dtype=np.float32)[:, None]
    i2 = np.arange(0, emb, 2).astype(np.float32)
    enc[:, 0::2] = np.sin(pos / 10000 ** (i2 / emb))
    enc[:, 1::2] = np.cos(pos / 10000 ** (i2 / emb))
    return enc


_POS = _pos_table(_EMB, _SEQ)


def _sc_body(table_hbm, x_hbm, pos_hbm, out_hbm, idx_v, rows_v, pos_v, sem):
    wid = lax.axis_index("s") * _NC + lax.axis_index("c")
    b = wid // 2
    t_half = (wid % 2) * _PER_W

    @pl.loop(0, _NCHUNK)
    def _chunk(c):
        t = t_half + c * _C
        pltpu.sync_copy(x_hbm.at[b, pl.ds(t, _C)], idx_v)
        pos_cp = pltpu.async_copy(pos_hbm.at[pl.ds(t, _C), :], pos_v, sem)

        @pl.loop(0, _C // _L)
        def _fire(g):
            rv = idx_v[pl.ds(g * _L, _L)]
            for j in range(_L):
                i = g * _L + j
                pltpu.async_copy(table_hbm.at[rv[j] & 0, :], rows_v.at[i], sem)

        pos_cp.wait()

        @pl.loop(0, _C)
        def _drain_rows(i):
            pltpu.make_async_copy(table_hbm.at[0, :], rows_v.at[i], sem).wait()

        @pl.loop(0, _C)
        def _add(i):
            for k in range(_EMB // _L):
                sl = pl.ds(k * _L, _L)
                rows_v[i, sl] = rows_v[i, sl] + pos_v[i, sl]

        pltpu.sync_copy(rows_v, out_hbm.at[b, pl.ds(t, _C), :])


@jax.jit
def _pos_ntok(x, table):
    mesh = plsc.VectorSubcoreMesh(core_axis_name="c", subcore_axis_name="s")
    fn = pl.kernel(
        _sc_body,
        out_type=jax.ShapeDtypeStruct((_BATCH, _SEQ, _EMB), jnp.float32),
        mesh=mesh,
        scratch_types=[
            pltpu.VMEM((_C,), jnp.int32),
            pltpu.VMEM((_C, _EMB), jnp.float32),
            pltpu.VMEM((_C, _EMB), jnp.float32),
            pltpu.SemaphoreType.DMA,
        ],
    )
    return fn(table, x, jnp.asarray(_POS))


def kernel(x, table):
    return _pos_ntok(x, table)


# speedup vs baseline: 4.1682x; 4.1682x over previous
import jax
import jax.numpy as jnp
import numpy as np
from jax import lax
from jax.experimental import pallas as pl
from jax.experimental.pallas import tpu as pltpu
from jax.experimental.pallas import tpu_sc as plsc

_VOCAB = 1000000
_EMB = 64
_BATCH = 16
_SEQ = 2048

_NC, _NS, _L = 2, 16, 16
_NW = _NC * _NS
_PER_W = _BATCH * _SEQ // _NW
_C = 128
_NCHUNK = _PER_W // _C


def _pos_table(emb, seq):
    enc = np.zeros((seq, emb), dtype=np.float32)
    pos = np.arange(0.0, seq, dtype=np.float32)[:, None]
    i2 = np.arange(0, emb, 2).astype(np.float32)
    enc[:, 0::2] = np.sin(pos / 10000 ** (i2 / emb))
    enc[:, 1::2] = np.cos(pos / 10000 ** (i2 / emb))
    return enc


_POS = _pos_table(_EMB, _SEQ)


def _sc_body(table_hbm, x_hbm, pos_hbm, out_hbm, idx2, rows2, pos2,
             sem_r0, sem_r1, sem_s0, sem_s1):
    wid = lax.axis_index("s") * _NC + lax.axis_index("c")
    b = wid // 2
    t_half = (wid % 2) * _PER_W
    sem_r = (sem_r0, sem_r1)
    sem_s = (sem_s0, sem_s1)

    def load_and_fire(c):
        p = c % 2
        t = t_half + c * _C
        pltpu.sync_copy(x_hbm.at[b, pl.ds(t, _C)], idx2.at[p])
        pltpu.async_copy(pos_hbm.at[pl.ds(t, _C), :], pos2.at[p], sem_r[p])

        @pl.loop(0, _C // _L)
        def _fire(g):
            rv = idx2[p, pl.ds(g * _L, _L)]
            for j in range(_L):
                pltpu.async_copy(
                    table_hbm.at[rv[j], :], rows2.at[p, g * _L + j], sem_r[p]
                )

    load_and_fire(0)
    for c in range(_NCHUNK):
        p = c % 2
        t = t_half + c * _C
        if c + 1 < _NCHUNK:
            if c >= 1:
                pltpu.make_async_copy(
                    rows2.at[1 - p], out_hbm.at[b, pl.ds(t, _C), :],
                    sem_s[1 - p],
                ).wait()
            load_and_fire(c + 1)

        pltpu.make_async_copy(
            pos_hbm.at[pl.ds(t, _C), :], pos2.at[p], sem_r[p]
        ).wait()
        pltpu.make_async_copy(
            table_hbm.at[pl.ds(0, _C), :], rows2.at[p], sem_r[p]
        ).wait()

        @pl.loop(0, _C)
        def _add(i):
            for k in range(_EMB // _L):
                sl = pl.ds(k * _L, _L)
                rows2[p, i, sl] = rows2[p, i, sl] + pos2[p, i, sl]

        pltpu.async_copy(rows2.at[p], out_hbm.at[b, pl.ds(t, _C), :], sem_s[p])

    pltpu.make_async_copy(
        rows2.at[0], out_hbm.at[b, pl.ds(t_half, _C), :], sem_s[0]
    ).wait()
    pltpu.make_async_copy(
        rows2.at[1], out_hbm.at[b, pl.ds(t_half, _C), :], sem_s[1]
    ).wait()


@jax.jit
def _pos_ntok(x, table):
    mesh = plsc.VectorSubcoreMesh(core_axis_name="c", subcore_axis_name="s")
    fn = pl.kernel(
        _sc_body,
        out_type=jax.ShapeDtypeStruct((_BATCH, _SEQ, _EMB), jnp.float32),
        mesh=mesh,
        scratch_types=[
            pltpu.VMEM((2, _C), jnp.int32),
            pltpu.VMEM((2, _C, _EMB), jnp.float32),
            pltpu.VMEM((2, _C, _EMB), jnp.float32),
            pltpu.SemaphoreType.DMA,
            pltpu.SemaphoreType.DMA,
            pltpu.SemaphoreType.DMA,
            pltpu.SemaphoreType.DMA,
        ],
    )
    return fn(table, x, jnp.asarray(_POS))


def kernel(x, table):
    return _pos_ntok(x, table)
